# baseline (speedup 1.0000x reference)
import functools

import jax
import jax.numpy as jnp
from jax import lax
from jax.experimental import pallas as pl
from jax.experimental.pallas import tpu as pltpu
from jax.experimental.pallas import tpu_sc as plsc

BETA_ = 0.05
KNN_ = 6
NEG_ = -1e30


def _oem(a, b):
    n, m = len(a), len(b)
    if n == 0:
        return list(b)
    if m == 0:
        return list(a)
    if n == 1 and m == 1:
        return [('max', a[0], b[0]), ('min', a[0], b[0])]
    e = _oem(a[0::2], b[0::2])
    o = _oem(a[1::2], b[1::2])
    z = [e[0]]
    i = 1
    while i < len(e) and i - 1 < len(o):
        z.append(('max', o[i - 1], e[i]))
        z.append(('min', o[i - 1], e[i]))
        i += 1
    z.extend(e[i:])
    z.extend(o[i - 1:])
    return z


def _sortnet(vals):
    if len(vals) == 1:
        return list(vals)
    h = len(vals) // 2
    return _oem(_sortnet(vals[:h]), _sortnet(vals[h:]))


def _emit(node, memo):
    if not isinstance(node, tuple):
        return node
    key = id(node)
    if key not in memo:
        op, x, y = node
        vx = _emit(x, memo)
        vy = _emit(y, memo)
        memo[key] = jnp.maximum(vx, vy) if op == 'max' else jnp.minimum(vx, vy)
    return memo[key]


def _top6_update(t6, grp):
    merged = _oem(t6, _sortnet(grp))[:KNN_]
    memo = {}
    return [_emit(nd, memo) for nd in merged]


def _body(x_ref, em_ref, tvp_ref, out_ref, sl_ref, t6_ref,
          *, B, C, Bt, Ct, nT):
    j = pl.program_id(0)
    b = pl.program_id(1)
    bs = pl.ds(b * Bt, Bt)

    @pl.when(j == 0)
    def _init():
        sl_ref[bs, :] = jnp.zeros((Bt, 128), jnp.float32)
        for i in range(KNN_):
            t6_ref[i, bs, :] = jnp.full((Bt, 128), -3e38, jnp.float32)

    pad = (j * Ct + lax.broadcasted_iota(jnp.int32, (1, Ct), 1)) >= C
    x = x_ref[bs, :]

    t6 = [t6_ref[i, bs, :] for i in range(KNN_)]
    sl = sl_ref[bs, :]
    nchunk = Ct // 128
    for c0 in range(0, nchunk, 4):
        emg = em_ref[pl.ds(c0 * 128, 512), :]
        rawg = lax.dot_general(x, emg, (((1,), (1,)), ((), ())),
                               preferred_element_type=jnp.float32)
        grp = [jnp.where(pad[:, (c0 + k) * 128:(c0 + k + 1) * 128], NEG_,
                         rawg[:, k * 128:(k + 1) * 128])
               for k in range(4)]
        ml_old = t6[0]
        t6 = _top6_update(t6, grp)
        ml = t6[0]
        acc = jnp.exp(grp[0] - ml)
        for k in range(1, 4):
            acc = acc + jnp.exp(grp[k] - ml)
        sl = sl * jnp.exp(ml_old - ml) + acc
    for i in range(KNN_):
        t6_ref[i, bs, :] = t6[i]
    sl_ref[bs, :] = sl

    @pl.when(j == nT - 1)
    def _epilogue():
        ml = t6[0]
        mrow = jnp.max(ml, axis=1, keepdims=True)
        srow = jnp.sum(sl * jnp.exp(ml - mrow), axis=1, keepdims=True)
        lse = jnp.log(srow) + mrow
        xx = jnp.concatenate(t6, axis=1)
        s6 = jnp.zeros((Bt, 1), jnp.float32)
        m = None
        for i in range(KNN_):
            m = jnp.max(xx, axis=1, keepdims=True)
            s6 = s6 + m
            if i < KNN_ - 1:
                xx = jnp.where(xx == m, NEG_, xx)
        v6 = m
        tv = jnp.sum(tvp_ref[...], axis=1, keepdims=True)
        in6 = (tv >= v6).astype(jnp.float32)
        w = 2.0 - in6 * (1.0 / KNN_)
        loss = lse * w - ((s6 - tv * in6) * (1.0 / KNN_) + tv)
        part = jnp.sum(loss, axis=0, keepdims=True) * (1.0 / B)
        prev = jnp.where(b == 0, jnp.zeros((1, 1), jnp.float32), out_ref[...])
        out_ref[...] = prev + part


def _make_call(B, F, C, Bt, Ct):
    nT = (C + Ct - 1) // Ct
    nB = B // Bt
    return pl.pallas_call(
        functools.partial(_body, B=B, C=C, Bt=Bt, Ct=Ct, nT=nT),
        grid=(nT, nB),
        in_specs=[
            pl.BlockSpec((B, F), lambda j, b: (0, 0)),
            pl.BlockSpec((Ct, F), lambda j, b: (j, 0)),
            pl.BlockSpec((Bt, 16), lambda j, b: (b, 0)),
        ],
        out_specs=pl.BlockSpec((1, 1), lambda j, b: (0, 0)),
        out_shape=jax.ShapeDtypeStruct((1, 1), jnp.float32),
        scratch_shapes=[
            pltpu.VMEM((B, 128), jnp.float32),
            pltpu.VMEM((KNN_, B, 128), jnp.float32),
        ],
    )


def _make_sc_tv(B, F):
    info = plsc.get_sparse_core_info()
    nc, ns = info.num_cores, info.num_subcores
    nw = nc * ns
    bpw = B // nw
    mesh = plsc.VectorSubcoreMesh(core_axis_name="c", subcore_axis_name="s")

    @functools.partial(
        pl.kernel, mesh=mesh,
        out_type=jax.ShapeDtypeStruct((B, 16), jnp.float32),
        scratch_types=[
            pltpu.VMEM((bpw,), jnp.int32),
            pltpu.VMEM((bpw, F), jnp.float32),
            pltpu.VMEM((bpw, F), jnp.float32),
            pltpu.VMEM((bpw, 16), jnp.float32),
            pltpu.SemaphoreType.DMA,
        ],
    )
    def k(x_hbm, t_hbm, em_hbm, out_hbm, idx_v, rows_v, x_v, tv_v, sem):
        wid = lax.axis_index("s") * nc + lax.axis_index("c")
        base = wid * bpw
        pltpu.sync_copy(t_hbm.at[pl.ds(base, bpw)], idx_v)
        pltpu.async_copy(em_hbm.at[idx_v], rows_v, sem).wait()
        pltpu.sync_copy(x_hbm.at[pl.ds(base, bpw)], x_v)
        for r in range(bpw):
            acc = rows_v[r, 0:16] * x_v[r, 0:16]
            for cc in range(1, F // 16):
                acc = acc + (rows_v[r, cc * 16:(cc + 1) * 16]
                             * x_v[r, cc * 16:(cc + 1) * 16])
            tv_v[r, :] = acc
        pltpu.sync_copy(tv_v, out_hbm.at[pl.ds(base, bpw)])

    return k


def kernel(inputs, targets, em, epoch):
    B, F = inputs.shape
    C = em.shape[0]
    Bt = min(B, 256)
    Ct = 4096
    x = inputs * (1.0 / BETA_)
    tvp = _make_sc_tv(B, F)(x, targets, em)
    out = _make_call(B, F, C, Bt, Ct)(x, em, tvp)
    return out[0, 0]


# speedup vs baseline: 1.0437x; 1.0437x over previous
import functools

import jax
import jax.numpy as jnp
from jax import lax
from jax.experimental import pallas as pl
from jax.experimental.pallas import tpu as pltpu
from jax.experimental.pallas import tpu_sc as plsc

BETA_ = 0.05
KNN_ = 6
NEG_ = -1e30


def _oem(a, b):
    n, m = len(a), len(b)
    if n == 0:
        return list(b)
    if m == 0:
        return list(a)
    if n == 1 and m == 1:
        return [('max', a[0], b[0]), ('min', a[0], b[0])]
    e = _oem(a[0::2], b[0::2])
    o = _oem(a[1::2], b[1::2])
    z = [e[0]]
    i = 1
    while i < len(e) and i - 1 < len(o):
        z.append(('max', o[i - 1], e[i]))
        z.append(('min', o[i - 1], e[i]))
        i += 1
    z.extend(e[i:])
    z.extend(o[i - 1:])
    return z


def _sortnet(vals):
    if len(vals) == 1:
        return list(vals)
    h = len(vals) // 2
    return _oem(_sortnet(vals[:h]), _sortnet(vals[h:]))


def _emit(node, memo):
    if not isinstance(node, tuple):
        return node
    key = id(node)
    if key not in memo:
        op, x, y = node
        vx = _emit(x, memo)
        vy = _emit(y, memo)
        memo[key] = jnp.maximum(vx, vy) if op == 'max' else jnp.minimum(vx, vy)
    return memo[key]


def _top6_update(t6, grp):
    merged = _oem(t6, _sortnet(grp))[:KNN_]
    memo = {}
    return [_emit(nd, memo) for nd in merged]


def _body(x_ref, em_ref, tvp_ref, out_ref, m_ref, s_ref, t6_ref,
          *, B, C, Bt, Ct, nT):
    j = pl.program_id(0)
    b = pl.program_id(1)
    bs = pl.ds(b * Bt, Bt)

    @pl.when(j == 0)
    def _init():
        m_ref[bs, :] = jnp.zeros((Bt, 1), jnp.float32)
        s_ref[bs, :] = jnp.zeros((Bt, 1), jnp.float32)
        for i in range(KNN_):
            t6_ref[i, bs, :] = jnp.full((Bt, 128), -3e38, jnp.float32)

    pad = (j * Ct + lax.broadcasted_iota(jnp.int32, (1, Ct), 1)) >= C
    x = x_ref[bs, :]

    t6 = [t6_ref[i, bs, :] for i in range(KNN_)]
    nchunk = Ct // 128
    vals = []
    for c0 in range(0, nchunk, 4):
        emg = em_ref[pl.ds(c0 * 128, 512), :]
        rawg = lax.dot_general(x, emg, (((1,), (1,)), ((), ())),
                               preferred_element_type=jnp.float32)
        grp = [jnp.where(pad[:, (c0 + k) * 128:(c0 + k + 1) * 128], NEG_,
                         rawg[:, k * 128:(k + 1) * 128])
               for k in range(4)]
        vals.extend(grp)
        t6 = _top6_update(t6, grp)
    for i in range(KNN_):
        t6_ref[i, bs, :] = t6[i]

    mn = jnp.max(t6[0], axis=1, keepdims=True)
    mo = m_ref[bs, :]
    p = jnp.zeros((Bt, 128), jnp.float32)
    for v in vals:
        p = p + jnp.exp(v - mn)
    s_ref[bs, :] = (s_ref[bs, :] * jnp.exp(jnp.minimum(mo - mn, 0.0))
                    + jnp.sum(p, axis=1, keepdims=True))
    m_ref[bs, :] = mn

    @pl.when(j == nT - 1)
    def _epilogue():
        lse = jnp.log(s_ref[bs, :]) + m_ref[bs, :]
        xx = jnp.concatenate(t6, axis=1)
        s6 = jnp.zeros((Bt, 1), jnp.float32)
        m = None
        for i in range(KNN_):
            m = jnp.max(xx, axis=1, keepdims=True)
            s6 = s6 + m
            if i < KNN_ - 1:
                xx = jnp.where(xx == m, NEG_, xx)
        v6 = m
        tv = jnp.sum(tvp_ref[...], axis=1, keepdims=True)
        in6 = (tv >= v6).astype(jnp.float32)
        w = 2.0 - in6 * (1.0 / KNN_)
        loss = lse * w - ((s6 - tv * in6) * (1.0 / KNN_) + tv)
        part = jnp.sum(loss, axis=0, keepdims=True) * (1.0 / B)
        prev = jnp.where(b == 0, jnp.zeros((1, 1), jnp.float32), out_ref[...])
        out_ref[...] = prev + part


def _make_call(B, F, C, Bt, Ct):
    nT = (C + Ct - 1) // Ct
    nB = B // Bt
    return pl.pallas_call(
        functools.partial(_body, B=B, C=C, Bt=Bt, Ct=Ct, nT=nT),
        grid=(nT, nB),
        in_specs=[
            pl.BlockSpec((B, F), lambda j, b: (0, 0)),
            pl.BlockSpec((Ct, F), lambda j, b: (j, 0)),
            pl.BlockSpec((Bt, 16), lambda j, b: (b, 0)),
        ],
        out_specs=pl.BlockSpec((1, 1), lambda j, b: (0, 0)),
        out_shape=jax.ShapeDtypeStruct((1, 1), jnp.float32),
        scratch_shapes=[
            pltpu.VMEM((B, 1), jnp.float32),
            pltpu.VMEM((B, 1), jnp.float32),
            pltpu.VMEM((KNN_, B, 128), jnp.float32),
        ],
    )


def _make_sc_tv(B, F):
    info = plsc.get_sparse_core_info()
    nc, ns = info.num_cores, info.num_subcores
    nw = nc * ns
    bpw = B // nw
    mesh = plsc.VectorSubcoreMesh(core_axis_name="c", subcore_axis_name="s")

    @functools.partial(
        pl.kernel, mesh=mesh,
        out_type=jax.ShapeDtypeStruct((B, 16), jnp.float32),
        scratch_types=[
            pltpu.VMEM((bpw,), jnp.int32),
            pltpu.VMEM((bpw, F), jnp.float32),
            pltpu.VMEM((bpw, F), jnp.float32),
            pltpu.VMEM((bpw, 16), jnp.float32),
            pltpu.SemaphoreType.DMA,
        ],
    )
    def k(x_hbm, t_hbm, em_hbm, out_hbm, idx_v, rows_v, x_v, tv_v, sem):
        wid = lax.axis_index("s") * nc + lax.axis_index("c")
        base = wid * bpw
        pltpu.sync_copy(t_hbm.at[pl.ds(base, bpw)], idx_v)
        pltpu.async_copy(em_hbm.at[idx_v], rows_v, sem).wait()
        pltpu.sync_copy(x_hbm.at[pl.ds(base, bpw)], x_v)
        for r in range(bpw):
            acc = rows_v[r, 0:16] * x_v[r, 0:16]
            for cc in range(1, F // 16):
                acc = acc + (rows_v[r, cc * 16:(cc + 1) * 16]
                             * x_v[r, cc * 16:(cc + 1) * 16])
            tv_v[r, :] = acc
        pltpu.sync_copy(tv_v, out_hbm.at[pl.ds(base, bpw)])

    return k


def kernel(inputs, targets, em, epoch):
    B, F = inputs.shape
    C = em.shape[0]
    Bt = min(B, 256)
    Ct = 4096
    x = inputs * (1.0 / BETA_)
    tvp = _make_sc_tv(B, F)(x, targets, em)
    out = _make_call(B, F, C, Bt, Ct)(x, em, tvp)
    return out[0, 0]


# speedup vs baseline: 1.0533x; 1.0091x over previous
import functools

import jax
import jax.numpy as jnp
from jax import lax
from jax.experimental import pallas as pl
from jax.experimental.pallas import tpu as pltpu
from jax.experimental.pallas import tpu_sc as plsc

BETA_ = 0.05
KNN_ = 6
NEG_ = -1e30
LOG2E_ = 1.4426950408889634
LN2_ = 0.6931471805599453


def _oem(a, b):
    n, m = len(a), len(b)
    if n == 0:
        return list(b)
    if m == 0:
        return list(a)
    if n == 1 and m == 1:
        return [('max', a[0], b[0]), ('min', a[0], b[0])]
    e = _oem(a[0::2], b[0::2])
    o = _oem(a[1::2], b[1::2])
    z = [e[0]]
    i = 1
    while i < len(e) and i - 1 < len(o):
        z.append(('max', o[i - 1], e[i]))
        z.append(('min', o[i - 1], e[i]))
        i += 1
    z.extend(e[i:])
    z.extend(o[i - 1:])
    return z


def _sortnet(vals):
    if len(vals) == 1:
        return list(vals)
    h = len(vals) // 2
    return _oem(_sortnet(vals[:h]), _sortnet(vals[h:]))


def _emit(node, memo):
    if not isinstance(node, tuple):
        return node
    key = id(node)
    if key not in memo:
        op, x, y = node
        vx = _emit(x, memo)
        vy = _emit(y, memo)
        memo[key] = jnp.maximum(vx, vy) if op == 'max' else jnp.minimum(vx, vy)
    return memo[key]


def _top6_update(t6, grp):
    merged = _oem(t6, _sortnet(grp))[:KNN_]
    memo = {}
    return [_emit(nd, memo) for nd in merged]


def _body(x_ref, em_ref, tvp_ref, out_ref, sl_ref, t6_ref,
          *, B, C, Bt, Ct, nT):
    j = pl.program_id(0)
    b = pl.program_id(1)
    bs = pl.ds(b * Bt, Bt)

    @pl.when(j == 0)
    def _init():
        sl_ref[bs, :] = jnp.zeros((Bt, 128), jnp.float32)
        for i in range(KNN_):
            t6_ref[i, bs, :] = jnp.full((Bt, 128), -3e38, jnp.float32)

    pad = (j * Ct + lax.broadcasted_iota(jnp.int32, (1, Ct), 1)) >= C
    x = x_ref[bs, :]

    t6 = [t6_ref[i, bs, :] for i in range(KNN_)]
    p = sl_ref[bs, :]
    mn = jnp.max(t6[0], axis=1, keepdims=True)
    nchunk = Ct // 128
    for c0 in range(0, nchunk, 4):
        emg = em_ref[pl.ds(c0 * 128, 512), :]
        rawg = lax.dot_general(x, emg, (((1,), (1,)), ((), ())),
                               preferred_element_type=jnp.float32)
        grp = [jnp.where(pad[:, (c0 + k) * 128:(c0 + k + 1) * 128], NEG_,
                         rawg[:, k * 128:(k + 1) * 128])
               for k in range(4)]
        t6 = _top6_update(t6, grp)
        mo = mn
        mn = jnp.max(t6[0], axis=1, keepdims=True)
        scale = jnp.exp2(jnp.minimum(mo - mn, 0.0))
        acc = (jnp.exp2(grp[0] - mn) + jnp.exp2(grp[1] - mn)
               + jnp.exp2(grp[2] - mn) + jnp.exp2(grp[3] - mn))
        p = p * scale + acc
    for i in range(KNN_):
        t6_ref[i, bs, :] = t6[i]
    sl_ref[bs, :] = p

    @pl.when(j == nT - 1)
    def _epilogue():
        srow = jnp.sum(p, axis=1, keepdims=True)
        lse = jnp.log(srow) + mn * LN2_
        xx = jnp.concatenate(t6, axis=1)
        s6 = jnp.zeros((Bt, 1), jnp.float32)
        m = None
        for i in range(KNN_):
            m = jnp.max(xx, axis=1, keepdims=True)
            s6 = s6 + m
            if i < KNN_ - 1:
                xx = jnp.where(xx == m, NEG_, xx)
        v6 = m
        tv = jnp.sum(tvp_ref[...], axis=1, keepdims=True)
        in6 = (tv >= v6).astype(jnp.float32)
        w = 2.0 - in6 * (1.0 / KNN_)
        loss = lse * w - LN2_ * ((s6 - tv * in6) * (1.0 / KNN_) + tv)
        part = jnp.sum(loss, axis=0, keepdims=True) * (1.0 / B)
        prev = jnp.where(b == 0, jnp.zeros((1, 1), jnp.float32), out_ref[...])
        out_ref[...] = prev + part


def _make_call(B, F, C, Bt, Ct):
    nT = (C + Ct - 1) // Ct
    nB = B // Bt
    return pl.pallas_call(
        functools.partial(_body, B=B, C=C, Bt=Bt, Ct=Ct, nT=nT),
        grid=(nT, nB),
        in_specs=[
            pl.BlockSpec((B, F), lambda j, b: (0, 0)),
            pl.BlockSpec((Ct, F), lambda j, b: (j, 0)),
            pl.BlockSpec((Bt, 16), lambda j, b: (b, 0)),
        ],
        out_specs=pl.BlockSpec((1, 1), lambda j, b: (0, 0)),
        out_shape=jax.ShapeDtypeStruct((1, 1), jnp.float32),
        scratch_shapes=[
            pltpu.VMEM((B, 128), jnp.float32),
            pltpu.VMEM((KNN_, B, 128), jnp.float32),
        ],
    )


def _make_sc_tv(B, F):
    info = plsc.get_sparse_core_info()
    nc, ns = info.num_cores, info.num_subcores
    nw = nc * ns
    bpw = B // nw
    mesh = plsc.VectorSubcoreMesh(core_axis_name="c", subcore_axis_name="s")

    @functools.partial(
        pl.kernel, mesh=mesh,
        out_type=jax.ShapeDtypeStruct((B, 16), jnp.float32),
        scratch_types=[
            pltpu.VMEM((bpw,), jnp.int32),
            pltpu.VMEM((bpw, F), jnp.float32),
            pltpu.VMEM((bpw, F), jnp.float32),
            pltpu.VMEM((bpw, 16), jnp.float32),
            pltpu.SemaphoreType.DMA,
        ],
    )
    def k(x_hbm, t_hbm, em_hbm, out_hbm, idx_v, rows_v, x_v, tv_v, sem):
        wid = lax.axis_index("s") * nc + lax.axis_index("c")
        base = wid * bpw
        pltpu.sync_copy(t_hbm.at[pl.ds(base, bpw)], idx_v)
        pltpu.async_copy(em_hbm.at[idx_v], rows_v, sem).wait()
        pltpu.sync_copy(x_hbm.at[pl.ds(base, bpw)], x_v)
        for r in range(bpw):
            acc = rows_v[r, 0:16] * x_v[r, 0:16]
            for cc in range(1, F // 16):
                acc = acc + (rows_v[r, cc * 16:(cc + 1) * 16]
                             * x_v[r, cc * 16:(cc + 1) * 16])
            tv_v[r, :] = acc
        pltpu.sync_copy(tv_v, out_hbm.at[pl.ds(base, bpw)])

    return k


def kernel(inputs, targets, em, epoch):
    B, F = inputs.shape
    C = em.shape[0]
    Bt = min(B, 256)
    Ct = 4096
    x = inputs * (LOG2E_ / BETA_)
    tvp = _make_sc_tv(B, F)(x, targets, em)
    out = _make_call(B, F, C, Bt, Ct)(x, em, tvp)
    return out[0, 0]


# speedup vs baseline: 1.0848x; 1.0299x over previous
import functools

import jax
import jax.numpy as jnp
from jax import lax
from jax.experimental import pallas as pl
from jax.experimental.pallas import tpu as pltpu
from jax.experimental.pallas import tpu_sc as plsc

BETA_ = 0.05
KNN_ = 6
NEG_ = -1e30
LOG2E_ = 1.4426950408889634
LN2_ = 0.6931471805599453


def _oem(a, b):
    n, m = len(a), len(b)
    if n == 0:
        return list(b)
    if m == 0:
        return list(a)
    if n == 1 and m == 1:
        return [('max', a[0], b[0]), ('min', a[0], b[0])]
    e = _oem(a[0::2], b[0::2])
    o = _oem(a[1::2], b[1::2])
    z = [e[0]]
    i = 1
    while i < len(e) and i - 1 < len(o):
        z.append(('max', o[i - 1], e[i]))
        z.append(('min', o[i - 1], e[i]))
        i += 1
    z.extend(e[i:])
    z.extend(o[i - 1:])
    return z


def _sortnet(vals):
    if len(vals) == 1:
        return list(vals)
    h = len(vals) // 2
    return _oem(_sortnet(vals[:h]), _sortnet(vals[h:]))


def _emit(node, memo):
    if not isinstance(node, tuple):
        return node
    key = id(node)
    if key not in memo:
        op, x, y = node
        vx = _emit(x, memo)
        vy = _emit(y, memo)
        memo[key] = jnp.maximum(vx, vy) if op == 'max' else jnp.minimum(vx, vy)
    return memo[key]


def _top6_update(t6, grp):
    merged = _oem(t6, _sortnet(grp))[:KNN_]
    memo = {}
    return [_emit(nd, memo) for nd in merged]


def _body(x_ref, em_ref, tvp_ref, out_ref, sl_ref, t6_ref,
          *, B, C, Bt, Ct, nT):
    j = pl.program_id(0)
    b = pl.program_id(1)
    bs = pl.ds(b * Bt, Bt)

    @pl.when(j == 0)
    def _init():
        sl_ref[bs, :] = jnp.zeros((Bt, 128), jnp.float32)
        for i in range(KNN_):
            t6_ref[i, bs, :] = jnp.full((Bt, 128), -3e38, jnp.float32)

    pad = (j * Ct + lax.broadcasted_iota(jnp.int32, (1, Ct), 1)) >= C
    x = x_ref[bs, :]

    t6 = [t6_ref[i, bs, :] for i in range(KNN_)]
    p = sl_ref[bs, :]
    mn = jnp.max(t6[0], axis=1, keepdims=True)
    nchunk = Ct // 128
    for c0 in range(0, nchunk, 4):
        emg = em_ref[pl.ds(c0 * 128, 512), :]
        rawg = lax.dot_general(x, emg, (((1,), (1,)), ((), ())),
                               preferred_element_type=jnp.float32)
        grp = [jnp.where(pad[:, (c0 + k) * 128:(c0 + k + 1) * 128], NEG_,
                         rawg[:, k * 128:(k + 1) * 128])
               for k in range(4)]
        t6 = _top6_update(t6, grp)
        mo = mn
        mn = jnp.max(t6[0], axis=1, keepdims=True)
        scale = jnp.exp2(jnp.minimum(mo - mn, 0.0))
        acc = (jnp.exp2(grp[0] - mn) + jnp.exp2(grp[1] - mn)
               + jnp.exp2(grp[2] - mn) + jnp.exp2(grp[3] - mn))
        p = p * scale + acc
    for i in range(KNN_):
        t6_ref[i, bs, :] = t6[i]
    sl_ref[bs, :] = p

    @pl.when(j == nT - 1)
    def _epilogue():
        srow = jnp.sum(p, axis=1, keepdims=True)
        lse = jnp.log(srow) + mn * LN2_
        xx = jnp.concatenate(t6, axis=1)
        s6 = jnp.zeros((Bt, 1), jnp.float32)
        m = None
        for i in range(KNN_):
            m = jnp.max(xx, axis=1, keepdims=True)
            s6 = s6 + m
            if i < KNN_ - 1:
                xx = jnp.where(xx == m, NEG_, xx)
        v6 = m
        tv = jnp.sum(tvp_ref[...], axis=1, keepdims=True)
        in6 = (tv >= v6).astype(jnp.float32)
        w = 2.0 - in6 * (1.0 / KNN_)
        loss = lse * w - LN2_ * ((s6 - tv * in6) * (1.0 / KNN_) + tv)
        part = jnp.sum(loss, axis=0, keepdims=True) * (1.0 / B)
        prev = jnp.where(b == 0, jnp.zeros((1, 1), jnp.float32), out_ref[...])
        out_ref[...] = prev + part


def _make_call(B, F, C, Bt, Ct):
    nT = (C + Ct - 1) // Ct
    nB = B // Bt
    return pl.pallas_call(
        functools.partial(_body, B=B, C=C, Bt=Bt, Ct=Ct, nT=nT),
        grid=(nT, nB),
        in_specs=[
            pl.BlockSpec((B, F), lambda j, b: (0, 0)),
            pl.BlockSpec((Ct, F), lambda j, b: (j, 0)),
            pl.BlockSpec((Bt, 16), lambda j, b: (b, 0)),
        ],
        out_specs=pl.BlockSpec((1, 1), lambda j, b: (0, 0)),
        out_shape=jax.ShapeDtypeStruct((1, 1), jnp.float32),
        scratch_shapes=[
            pltpu.VMEM((B, 128), jnp.float32),
            pltpu.VMEM((KNN_, B, 128), jnp.float32),
        ],
    )


def _make_sc_tv(B, F):
    info = plsc.get_sparse_core_info()
    nc, ns = info.num_cores, info.num_subcores
    nw = nc * ns
    bpw = B // nw
    mesh = plsc.VectorSubcoreMesh(core_axis_name="c", subcore_axis_name="s")

    @functools.partial(
        pl.kernel, mesh=mesh,
        out_type=jax.ShapeDtypeStruct((B, 16), jnp.float32),
        scratch_types=[
            pltpu.VMEM((bpw,), jnp.int32),
            pltpu.VMEM((bpw, F), jnp.float32),
            pltpu.VMEM((bpw, F), jnp.float32),
            pltpu.VMEM((bpw, 16), jnp.float32),
            pltpu.SemaphoreType.DMA,
        ],
    )
    def k(x_hbm, t_hbm, em_hbm, out_hbm, idx_v, rows_v, x_v, tv_v, sem):
        wid = lax.axis_index("s") * nc + lax.axis_index("c")
        base = wid * bpw
        pltpu.sync_copy(t_hbm.at[pl.ds(base, bpw)], idx_v)
        pltpu.async_copy(em_hbm.at[idx_v], rows_v, sem).wait()
        pltpu.sync_copy(x_hbm.at[pl.ds(base, bpw)], x_v)
        for r in range(bpw):
            acc = rows_v[r, 0:16] * x_v[r, 0:16]
            for cc in range(1, F // 16):
                acc = acc + (rows_v[r, cc * 16:(cc + 1) * 16]
                             * x_v[r, cc * 16:(cc + 1) * 16])
            tv_v[r, :] = acc
        pltpu.sync_copy(tv_v, out_hbm.at[pl.ds(base, bpw)])

    return k


def kernel(inputs, targets, em, epoch):
    B, F = inputs.shape
    C = em.shape[0]
    Bt = min(B, 512)
    Ct = 4096
    x = inputs * (LOG2E_ / BETA_)
    tvp = _make_sc_tv(B, F)(x, targets, em)
    out = _make_call(B, F, C, Bt, Ct)(x, em, tvp)
    return out[0, 0]


# speedup vs baseline: 1.2111x; 1.1164x over previous
import functools

import jax
import jax.numpy as jnp
from jax import lax
from jax.experimental import pallas as pl
from jax.experimental.pallas import tpu as pltpu
from jax.experimental.pallas import tpu_sc as plsc

BETA_ = 0.05
KNN_ = 6
NEG_ = -1e30
LOG2E_ = 1.4426950408889634
LN2_ = 0.6931471805599453


def _oem(a, b):
    n, m = len(a), len(b)
    if n == 0:
        return list(b)
    if m == 0:
        return list(a)
    if n == 1 and m == 1:
        return [('max', a[0], b[0]), ('min', a[0], b[0])]
    e = _oem(a[0::2], b[0::2])
    o = _oem(a[1::2], b[1::2])
    z = [e[0]]
    i = 1
    while i < len(e) and i - 1 < len(o):
        z.append(('max', o[i - 1], e[i]))
        z.append(('min', o[i - 1], e[i]))
        i += 1
    z.extend(e[i:])
    z.extend(o[i - 1:])
    return z


def _sortnet(vals):
    if len(vals) == 1:
        return list(vals)
    h = len(vals) // 2
    return _oem(_sortnet(vals[:h]), _sortnet(vals[h:]))


def _emit(node, memo):
    if not isinstance(node, tuple):
        return node
    key = id(node)
    if key not in memo:
        op, x, y = node
        vx = _emit(x, memo)
        vy = _emit(y, memo)
        memo[key] = jnp.maximum(vx, vy) if op == 'max' else jnp.minimum(vx, vy)
    return memo[key]


def _top6_update(t6, grp):
    merged = _oem(t6, _sortnet(grp))[:KNN_]
    memo = {}
    return [_emit(nd, memo) for nd in merged]


def _body(x_ref, em_ref, tvp_ref, out_ref, sl_ref, t6_ref,
          *, B, C, Bt, Ct, nT):
    j = pl.program_id(0)
    b = pl.program_id(1)
    bs = pl.ds(b * Bt, Bt)

    @pl.when(j == 0)
    def _init():
        sl_ref[bs, :] = jnp.zeros((Bt, 128), jnp.float32)
        for i in range(KNN_):
            t6_ref[i, bs, :] = jnp.full((Bt, 128), -3e38, jnp.float32)

    pad = (j * Ct + lax.broadcasted_iota(jnp.int32, (1, Ct), 1)) >= C
    x = x_ref[bs, :]

    t6 = [t6_ref[i, bs, :] for i in range(KNN_)]
    p = sl_ref[bs, :]
    mn = jnp.max(t6[0], axis=1, keepdims=True)
    nchunk = Ct // 128
    for c0 in range(0, nchunk, 4):
        emg = em_ref[pl.ds(c0 * 128, 512), :]
        rawg = lax.dot_general(x, emg, (((1,), (1,)), ((), ())),
                               preferred_element_type=jnp.float32)
        grp = [jnp.where(pad[:, (c0 + k) * 128:(c0 + k + 1) * 128], NEG_,
                         rawg[:, k * 128:(k + 1) * 128])
               for k in range(4)]
        t6 = _top6_update(t6, grp)
        mo = mn
        mn = jnp.max(t6[0], axis=1, keepdims=True)
        scale = jnp.exp2(jnp.minimum(mo - mn, 0.0))
        acc = (jnp.exp2(grp[0] - mn) + jnp.exp2(grp[1] - mn)
               + jnp.exp2(grp[2] - mn) + jnp.exp2(grp[3] - mn))
        p = p * scale + acc
    for i in range(KNN_):
        t6_ref[i, bs, :] = t6[i]
    sl_ref[bs, :] = p

    @pl.when(j == nT - 1)
    def _epilogue():
        srow = jnp.sum(p, axis=1, keepdims=True)
        lse = jnp.log(srow) + mn * LN2_
        xx = jnp.concatenate(t6, axis=1)
        s6 = jnp.zeros((Bt, 1), jnp.float32)
        m = None
        for i in range(KNN_):
            m = jnp.max(xx, axis=1, keepdims=True)
            s6 = s6 + m
            if i < KNN_ - 1:
                xx = jnp.where(xx == m, NEG_, xx)
        v6 = m
        tv = jnp.sum(tvp_ref[...], axis=1, keepdims=True)
        in6 = (tv >= v6).astype(jnp.float32)
        w = 2.0 - in6 * (1.0 / KNN_)
        loss = lse * w - LN2_ * ((s6 - tv * in6) * (1.0 / KNN_) + tv)
        part = jnp.sum(loss, axis=0, keepdims=True) * (1.0 / B)
        prev = jnp.where(b == 0, jnp.zeros((1, 1), jnp.float32), out_ref[...])
        out_ref[...] = prev + part


def _make_call(B, F, C, Bt, Ct):
    nT = (C + Ct - 1) // Ct
    nB = B // Bt
    return pl.pallas_call(
        functools.partial(_body, B=B, C=C, Bt=Bt, Ct=Ct, nT=nT),
        grid=(nT, nB),
        in_specs=[
            pl.BlockSpec((B, F), lambda j, b: (0, 0)),
            pl.BlockSpec((Ct, F), lambda j, b: (j, 0)),
            pl.BlockSpec((Bt, 16), lambda j, b: (b, 0)),
        ],
        out_specs=pl.BlockSpec((1, 1), lambda j, b: (0, 0)),
        out_shape=jax.ShapeDtypeStruct((1, 1), jnp.float32),
        scratch_shapes=[
            pltpu.VMEM((B, 128), jnp.float32),
            pltpu.VMEM((KNN_, B, 128), jnp.float32),
        ],
    )


def _make_sc_tv(B, F):
    info = plsc.get_sparse_core_info()
    nc, ns = info.num_cores, info.num_subcores
    nw = nc * ns
    bpw = B // nw
    mesh = plsc.VectorSubcoreMesh(core_axis_name="c", subcore_axis_name="s")

    @functools.partial(
        pl.kernel, mesh=mesh,
        out_type=jax.ShapeDtypeStruct((B, 16), jnp.float32),
        scratch_types=[
            pltpu.VMEM((bpw,), jnp.int32),
            pltpu.VMEM((bpw, F), jnp.float32),
            pltpu.VMEM((bpw, F), jnp.float32),
            pltpu.VMEM((bpw, 16), jnp.float32),
            pltpu.SemaphoreType.DMA,
        ],
    )
    def k(x_hbm, t_hbm, em_hbm, out_hbm, idx_v, rows_v, x_v, tv_v, sem):
        wid = lax.axis_index("s") * nc + lax.axis_index("c")
        base = wid * bpw
        pltpu.sync_copy(t_hbm.at[pl.ds(base, bpw)], idx_v)
        pltpu.async_copy(em_hbm.at[idx_v], rows_v, sem).wait()
        pltpu.sync_copy(x_hbm.at[pl.ds(base, bpw)], x_v)
        for r in range(bpw):
            acc = rows_v[r, 0:16] * x_v[r, 0:16]
            for cc in range(1, F // 16):
                acc = acc + (rows_v[r, cc * 16:(cc + 1) * 16]
                             * x_v[r, cc * 16:(cc + 1) * 16])
            tv_v[r, :] = acc
        pltpu.sync_copy(tv_v, out_hbm.at[pl.ds(base, bpw)])

    return k


def kernel(inputs, targets, em, epoch):
    B, F = inputs.shape
    C = em.shape[0]
    Bt = min(B, 1024)
    Ct = 4096
    x = inputs * (LOG2E_ / BETA_)
    tvp = _make_sc_tv(B, F)(x, targets, em)
    out = _make_call(B, F, C, Bt, Ct)(x, em, tvp)
    return out[0, 0]


# speedup vs baseline: 1.3065x; 1.0788x over previous
import functools

import jax
import jax.numpy as jnp
from jax import lax
from jax.experimental import pallas as pl
from jax.experimental.pallas import tpu as pltpu
from jax.experimental.pallas import tpu_sc as plsc

BETA_ = 0.05
KNN_ = 6
NEG_ = -1e30
LOG2E_ = 1.4426950408889634
LN2_ = 0.6931471805599453


def _oem(a, b):
    n, m = len(a), len(b)
    if n == 0:
        return list(b)
    if m == 0:
        return list(a)
    if n == 1 and m == 1:
        return [('max', a[0], b[0]), ('min', a[0], b[0])]
    e = _oem(a[0::2], b[0::2])
    o = _oem(a[1::2], b[1::2])
    z = [e[0]]
    i = 1
    while i < len(e) and i - 1 < len(o):
        z.append(('max', o[i - 1], e[i]))
        z.append(('min', o[i - 1], e[i]))
        i += 1
    z.extend(e[i:])
    z.extend(o[i - 1:])
    return z


def _sortnet(vals):
    if len(vals) == 1:
        return list(vals)
    h = len(vals) // 2
    return _oem(_sortnet(vals[:h]), _sortnet(vals[h:]))


def _emit(node, memo):
    if not isinstance(node, tuple):
        return node
    key = id(node)
    if key not in memo:
        op, x, y = node
        vx = _emit(x, memo)
        vy = _emit(y, memo)
        memo[key] = jnp.maximum(vx, vy) if op == 'max' else jnp.minimum(vx, vy)
    return memo[key]


def _top6_update(t6, grp):
    merged = _oem(t6, _sortnet(grp))[:KNN_]
    memo = {}
    return [_emit(nd, memo) for nd in merged]


def _body(x_ref, em_ref, tvp_ref, out_ref, sl_ref, t6_ref,
          *, B, C, Bt, Ct, nT):
    j = pl.program_id(0)
    b = pl.program_id(1)
    bs = pl.ds(b * Bt, Bt)

    @pl.when(j == 0)
    def _init():
        sl_ref[bs, :] = jnp.zeros((Bt, 128), jnp.float32)
        for i in range(KNN_):
            t6_ref[i, bs, :] = jnp.full((Bt, 128), -3e38, jnp.float32)

    x = x_ref[bs, :]

    def _sweep(masked):
        pad = (j * Ct + lax.broadcasted_iota(jnp.int32, (1, Ct), 1)) >= C
        t6 = [t6_ref[i, bs, :] for i in range(KNN_)]
        p = sl_ref[bs, :]
        mn = jnp.max(t6[0], axis=1, keepdims=True)
        nchunk = Ct // 128
        for c0 in range(0, nchunk, 4):
            emg = em_ref[pl.ds(c0 * 128, 512), :]
            rawg = lax.dot_general(x, emg, (((1,), (1,)), ((), ())),
                                   preferred_element_type=jnp.float32)
            grp = [rawg[:, k * 128:(k + 1) * 128] for k in range(4)]
            if masked:
                grp = [jnp.where(pad[:, (c0 + k) * 128:(c0 + k + 1) * 128],
                                 NEG_, g) for k, g in enumerate(grp)]
            t6 = _top6_update(t6, grp)
            mo = mn
            mn = jnp.max(t6[0], axis=1, keepdims=True)
            scale = jnp.exp2(jnp.minimum(mo - mn, 0.0))
            acc = (jnp.exp2(grp[0] - mn) + jnp.exp2(grp[1] - mn)
                   + jnp.exp2(grp[2] - mn) + jnp.exp2(grp[3] - mn))
            p = p * scale + acc
        for i in range(KNN_):
            t6_ref[i, bs, :] = t6[i]
        sl_ref[bs, :] = p
        return t6, p, mn

    @pl.when(j < nT - 1)
    def _main_tiles():
        _sweep(masked=False)

    @pl.when(j == nT - 1)
    def _epilogue():
        t6, p, mn = _sweep(masked=True)
        srow = jnp.sum(p, axis=1, keepdims=True)
        lse = jnp.log(srow) + mn * LN2_
        xx = jnp.concatenate(t6, axis=1)
        s6 = jnp.zeros((Bt, 1), jnp.float32)
        m = None
        for i in range(KNN_):
            m = jnp.max(xx, axis=1, keepdims=True)
            s6 = s6 + m
            if i < KNN_ - 1:
                xx = jnp.where(xx == m, NEG_, xx)
        v6 = m
        tv = jnp.sum(tvp_ref[...], axis=1, keepdims=True)
        in6 = (tv >= v6).astype(jnp.float32)
        w = 2.0 - in6 * (1.0 / KNN_)
        loss = lse * w - LN2_ * ((s6 - tv * in6) * (1.0 / KNN_) + tv)
        part = jnp.sum(loss, axis=0, keepdims=True) * (1.0 / B)
        prev = jnp.where(b == 0, jnp.zeros((1, 1), jnp.float32), out_ref[...])
        out_ref[...] = prev + part


def _make_call(B, F, C, Bt, Ct):
    nT = (C + Ct - 1) // Ct
    nB = B // Bt
    return pl.pallas_call(
        functools.partial(_body, B=B, C=C, Bt=Bt, Ct=Ct, nT=nT),
        grid=(nT, nB),
        in_specs=[
            pl.BlockSpec((B, F), lambda j, b: (0, 0)),
            pl.BlockSpec((Ct, F), lambda j, b: (j, 0)),
            pl.BlockSpec((Bt, 16), lambda j, b: (b, 0)),
        ],
        out_specs=pl.BlockSpec((1, 1), lambda j, b: (0, 0)),
        out_shape=jax.ShapeDtypeStruct((1, 1), jnp.float32),
        scratch_shapes=[
            pltpu.VMEM((B, 128), jnp.float32),
            pltpu.VMEM((KNN_, B, 128), jnp.float32),
        ],
    )


def _make_sc_tv(B, F):
    info = plsc.get_sparse_core_info()
    nc, ns = info.num_cores, info.num_subcores
    nw = nc * ns
    bpw = B // nw
    mesh = plsc.VectorSubcoreMesh(core_axis_name="c", subcore_axis_name="s")

    @functools.partial(
        pl.kernel, mesh=mesh,
        out_type=jax.ShapeDtypeStruct((B, 16), jnp.float32),
        scratch_types=[
            pltpu.VMEM((bpw,), jnp.int32),
            pltpu.VMEM((bpw, F), jnp.float32),
            pltpu.VMEM((bpw, F), jnp.float32),
            pltpu.VMEM((bpw, 16), jnp.float32),
            pltpu.SemaphoreType.DMA,
        ],
    )
    def k(x_hbm, t_hbm, em_hbm, out_hbm, idx_v, rows_v, x_v, tv_v, sem):
        wid = lax.axis_index("s") * nc + lax.axis_index("c")
        base = wid * bpw
        pltpu.sync_copy(t_hbm.at[pl.ds(base, bpw)], idx_v)
        pltpu.async_copy(em_hbm.at[idx_v], rows_v, sem).wait()
        pltpu.sync_copy(x_hbm.at[pl.ds(base, bpw)], x_v)
        for r in range(bpw):
            acc = rows_v[r, 0:16] * x_v[r, 0:16]
            for cc in range(1, F // 16):
                acc = acc + (rows_v[r, cc * 16:(cc + 1) * 16]
                             * x_v[r, cc * 16:(cc + 1) * 16])
            tv_v[r, :] = acc
        pltpu.sync_copy(tv_v, out_hbm.at[pl.ds(base, bpw)])

    return k


def kernel(inputs, targets, em, epoch):
    B, F = inputs.shape
    C = em.shape[0]
    Bt = min(B, 1024)
    Ct = 4096
    x = inputs * (LOG2E_ / BETA_)
    tvp = _make_sc_tv(B, F)(x, targets, em)
    out = _make_call(B, F, C, Bt, Ct)(x, em, tvp)
    return out[0, 0]
